# Initial kernel scaffold; baseline (speedup 1.0000x reference)
import jax
import jax.numpy as jnp
from jax.experimental import pallas as pl


def kernel(x, edge_index, W_enc1, b_enc1, W_enc2, b_enc2, W_msg, b_msg, W_upd, b_upd, W_r1, b_r1, W_r2, b_r2):
    raise NotImplementedError("write your pallas kernel here")


# speedup vs baseline: 3.9824x; 3.9824x over previous
import functools

import jax
import jax.numpy as jnp
from jax import lax
from jax.experimental import pallas as pl
from jax.experimental.pallas import tpu as pltpu
from jax.experimental.pallas import tpu_sc as plsc

N = 10000
E = 320000
D = 128
H = 128
L = 128
ITERS = 3

NC = 2
NS = 16
CH = 128
NW = NC * NS
CH_PER_W = -(-E // (CH * NW))
EPAD = CH_PER_W * CH * NW
NPAD = 10240

ROWS_PER_TILE = NPAD // NS
ZCHUNKS = ROWS_PER_TILE // CH


def _sc_edge_body(a_hbm, b_hbm, src_hbm, dst_hbm, out_hbm,
                  srcv, dstv, arows, brows, aggsh, sem_a, sem_b):
    cid = lax.axis_index("c")
    sid = lax.axis_index("s")
    w = sid * NC + cid

    zero = jnp.zeros((16,), jnp.float32)

    def zrow(e, carry):
        for j in range(H // 16):
            arows[e, pl.ds(j * 16, 16)] = zero
        return carry

    lax.fori_loop(0, CH, zrow, 0)
    r0 = sid * ROWS_PER_TILE
    for t in range(ZCHUNKS):
        pltpu.sync_copy(arows, aggsh.at[pl.ds(r0 + t * CH, CH)])
    plsc.subcore_barrier()

    def chunk(i, carry):
        base = (w * CH_PER_W + i) * CH
        pltpu.sync_copy(src_hbm.at[pl.ds(base, CH)], srcv)
        pltpu.sync_copy(dst_hbm.at[pl.ds(base, CH)], dstv)
        ca = pltpu.async_copy(a_hbm.at[srcv], arows, sem_a)
        cb = pltpu.async_copy(b_hbm.at[dstv], brows, sem_b)
        ca.wait()
        cb.wait()

        def edge(e, c2):
            for j in range(H // 16):
                s = pl.ds(j * 16, 16)
                arows[e, s] = jnp.maximum(arows[e, s] + brows[e, s], 0.0)
            return c2

        lax.fori_loop(0, CH, edge, 0)
        pltpu.sync_copy(arows, aggsh.at[dstv], add=True)
        return carry

    lax.fori_loop(0, CH_PER_W, chunk, 0)
    plsc.subcore_barrier()

    for t in range(ZCHUNKS):
        pltpu.sync_copy(aggsh.at[pl.ds(r0 + t * CH, CH)],
                        out_hbm.at[cid, pl.ds(r0 + t * CH, CH)])


@functools.lru_cache(maxsize=1)
def _sc_edge():
    return functools.partial(
        pl.kernel,
        mesh=plsc.VectorSubcoreMesh(core_axis_name="c", subcore_axis_name="s"),
        out_type=jax.ShapeDtypeStruct((NC, NPAD, H), jnp.float32),
        scratch_types=[
            pltpu.VMEM((CH,), jnp.int32),
            pltpu.VMEM((CH,), jnp.int32),
            pltpu.VMEM((CH, H), jnp.float32),
            pltpu.VMEM((CH, H), jnp.float32),
            pltpu.VMEM_SHARED((NPAD, H), jnp.float32),
            pltpu.SemaphoreType.DMA,
            pltpu.SemaphoreType.DMA,
        ],
    )(_sc_edge_body)


BLK = 1000
GRID = N // BLK


def _mm(a, b):
    return jnp.dot(a, b, preferred_element_type=jnp.float32)


def _enc_kernel(x_ref, w1, b1, w2, b2, wmt, wmb, bm, h_ref, a_ref, b_ref):
    h1 = jnp.maximum(_mm(x_ref[...], w1[...]) + b1[...], 0.0)
    h = jnp.maximum(_mm(h1, w2[...]) + b2[...], 0.0)
    h_ref[...] = h
    a_ref[...] = _mm(h, wmt[...]) + bm[...]
    b_ref[...] = _mm(h, wmb[...])


def _upd_kernel(h_ref, agg_ref, wut, wub, bu, wmt, wmb, bm,
                hn_ref, a_ref, b_ref):
    agg = agg_ref[0] + agg_ref[1]
    hn = jnp.maximum(_mm(h_ref[...], wut[...]) + _mm(agg, wub[...]) + bu[...],
                     0.0)
    hn_ref[...] = hn
    a_ref[...] = _mm(hn, wmt[...]) + bm[...]
    b_ref[...] = _mm(hn, wmb[...])


def _upd_last_kernel(h_ref, agg_ref, wut, wub, bu, wr1, br1, wr2, br2,
                     out_ref, gacc):
    i = pl.program_id(0)
    agg = agg_ref[0] + agg_ref[1]
    hn = jnp.maximum(_mm(h_ref[...], wut[...]) + _mm(agg, wub[...]) + bu[...],
                     0.0)
    part = jnp.sum(hn.reshape(BLK // 8, 8, H), axis=0)

    @pl.when(i == 0)
    def _init():
        gacc[...] = part

    @pl.when(i > 0)
    def _acc():
        gacc[...] = gacc[...] + part

    @pl.when(i == GRID - 1)
    def _readout():
        g = jnp.sum(gacc[...], axis=0, keepdims=True)
        lat = _mm(jnp.maximum(_mm(g, wr1[...]) + br1[...], 0.0), wr2[...])
        out_ref[...] = lat + br2[...]


def _row_spec():
    return pl.BlockSpec((BLK, H), lambda i: (i, 0))


def _full_spec(shape):
    return pl.BlockSpec(shape, lambda i: tuple(0 for _ in shape))


_NODE_SHAPE = jax.ShapeDtypeStruct((N, H), jnp.float32)

_enc_call = pl.pallas_call(
    _enc_kernel,
    grid=(GRID,),
    in_specs=[_row_spec()] + [_full_spec(s) for s in
                              [(D, H), (1, H), (H, H), (1, H),
                               (H, H), (H, H), (1, H)]],
    out_specs=[_row_spec(), _row_spec(), _row_spec()],
    out_shape=[_NODE_SHAPE, _NODE_SHAPE, _NODE_SHAPE],
)

_upd_call = pl.pallas_call(
    _upd_kernel,
    grid=(GRID,),
    in_specs=[_row_spec(),
              pl.BlockSpec((NC, BLK, H), lambda i: (0, i, 0))] +
             [_full_spec(s) for s in
              [(H, H), (H, H), (1, H), (H, H), (H, H), (1, H)]],
    out_specs=[_row_spec(), _row_spec(), _row_spec()],
    out_shape=[_NODE_SHAPE, _NODE_SHAPE, _NODE_SHAPE],
)

_upd_last_call = pl.pallas_call(
    _upd_last_kernel,
    grid=(GRID,),
    in_specs=[_row_spec(),
              pl.BlockSpec((NC, BLK, H), lambda i: (0, i, 0))] +
             [_full_spec(s) for s in
              [(H, H), (H, H), (1, H), (H, H), (1, H), (H, L), (1, L)]],
    out_specs=pl.BlockSpec((1, L), lambda i: (0, 0)),
    out_shape=jax.ShapeDtypeStruct((1, L), jnp.float32),
    scratch_shapes=[pltpu.VMEM((8, H), jnp.float32)],
)


def kernel(x, edge_index, W_enc1, b_enc1, W_enc2, b_enc2, W_msg, b_msg,
           W_upd, b_upd, W_r1, b_r1, W_r2, b_r2):
    src = edge_index[0].astype(jnp.int32)
    dst = edge_index[1].astype(jnp.int32)
    src_p = jnp.concatenate([src, jnp.zeros((EPAD - E,), jnp.int32)])
    dst_p = jnp.concatenate([dst, jnp.full((EPAD - E,), N, jnp.int32)])

    wmt, wmb = W_msg[:H], W_msg[H:]
    wut, wub = W_upd[:H], W_upd[H:]
    b1 = b_enc1.reshape(1, H)
    b2 = b_enc2.reshape(1, H)
    bm = b_msg.reshape(1, H)
    bu = b_upd.reshape(1, H)
    br1 = b_r1.reshape(1, H)
    br2 = b_r2.reshape(1, L)

    h, a, b = _enc_call(x, W_enc1, b1, W_enc2, b2, wmt, wmb, bm)
    for it in range(ITERS):
        aggp = _sc_edge()(a, b, src_p, dst_p)
        if it < ITERS - 1:
            h, a, b = _upd_call(h, aggp, wut, wub, bu, wmt, wmb, bm)
        else:
            lat = _upd_last_call(h, aggp, wut, wub, bu,
                                 W_r1, br1, W_r2, br2)
    return lat.reshape(L)
